# baseline (speedup 1.0000x reference)
import functools

import jax
import jax.numpy as jnp
from jax import lax
from jax.experimental import pallas as pl
from jax.experimental.pallas import tpu as pltpu
from jax.experimental.pallas import tpu_sc as plsc

_DIM = 128
_LANES = 16
_NC = 2
_NS = 16
_NW = _NC * _NS
_VPR = _DIM // _LANES

_SUB = 10000
_PADBLK = 10240


def _tc_inv_norm(weight):
    v = weight.shape[0]
    assert v % _SUB == 0
    nb = v // _SUB

    def body(w_ref, inv_ref):
        w = w_ref[...]
        ones = jnp.ones((_DIM, 1), jnp.float32)
        ss = lax.dot_general(w * w, ones, (((1,), (0,)), ((), ())),
                             preferred_element_type=jnp.float32)[:, 0]
        inv_ref[pl.ds(0, _SUB)] = lax.rsqrt(jnp.maximum(ss, 1e-24))

    return pl.pallas_call(
        body,
        grid=(nb,),
        in_specs=[pl.BlockSpec((_SUB, _DIM), lambda i: (i, 0))],
        out_specs=pl.BlockSpec((_PADBLK,), lambda i: (i,)),
        out_shape=jax.ShapeDtypeStruct((nb * _PADBLK,), jnp.float32),
    )(weight)


def _scale_rows(rows_ref, inv_ref, nrows):
    @plsc.parallel_loop(0, nrows // _LANES, step=1)
    def group_fn(g):
        iv = inv_ref[pl.ds(g * _LANES, _LANES)]
        for k in range(_LANES):
            r = g * _LANES + k
            inv = iv[k]
            for j in range(_VPR):
                sl = pl.ds(j * _LANES, _LANES)
                rows_ref[r, sl] = rows_ref[r, sl] * inv


def _make_sc_kernel(nb_out, l_out, chunk):
    n_idx = nb_out * l_out
    assert chunk % l_out == 0 and n_idx % (_NW * chunk) == 0
    assert chunk % _LANES == 0
    slabs_per_chunk = chunk // l_out
    per_w = n_idx // _NW
    nchunk = per_w // chunk
    assert nchunk % 2 == 0
    mesh = plsc.VectorSubcoreMesh(core_axis_name="c", subcore_axis_name="s")

    @functools.partial(
        pl.kernel,
        out_type=jax.ShapeDtypeStruct((nb_out, l_out, _DIM), jnp.float32),
        mesh=mesh,
        scratch_types=[
            pltpu.VMEM((chunk,), jnp.int32),
            pltpu.VMEM((chunk,), jnp.int32),
            pltpu.VMEM((chunk,), jnp.int32),
            pltpu.VMEM((chunk,), jnp.int32),
            pltpu.VMEM((chunk, _DIM), jnp.float32),
            pltpu.VMEM((chunk, _DIM), jnp.float32),
            pltpu.VMEM((chunk,), jnp.float32),
            pltpu.VMEM((chunk,), jnp.float32),
            pltpu.SemaphoreType.DMA,
            pltpu.SemaphoreType.DMA,
            pltpu.SemaphoreType.DMA,
            pltpu.SemaphoreType.DMA,
            pltpu.SemaphoreType.DMA,
            pltpu.SemaphoreType.DMA,
        ],
        compiler_params=pltpu.CompilerParams(needs_layout_passes=False,
                                             use_tc_tiling_on_sc=True),
    )
    def sc_kernel(idx_hbm, w_hbm, inv_hbm, out_hbm, idx_a, idx_b, idxp_a,
                  idxp_b, rows_a, rows_b, inv_a, inv_b, sem_a, sem_b,
                  isem_a, isem_b, osem_a, osem_b):
        wid = lax.axis_index("s") * _NC + lax.axis_index("c")
        base = wid * per_w
        idx_bufs = (idx_a, idx_b)
        idxp_bufs = (idxp_a, idxp_b)
        row_bufs = (rows_a, rows_b)
        inv_bufs = (inv_a, inv_b)
        sems = (sem_a, sem_b)
        isems = (isem_a, isem_b)
        osems = (osem_a, osem_b)

        def fetch(c, b):
            off = base + c * chunk
            pltpu.sync_copy(idx_hbm.at[pl.ds(off, chunk)], idx_bufs[b])

            @plsc.parallel_loop(0, chunk // _LANES, step=1, unroll=2)
            def pad_fn(t):
                sl = pl.ds(t * _LANES, _LANES)
                iv = idx_bufs[b][sl]
                idxp_bufs[b][sl] = iv + (_PADBLK - _SUB) * (iv // _SUB)

            pltpu.async_copy(w_hbm.at[idx_bufs[b]], row_bufs[b], sems[b])
            pltpu.async_copy(inv_hbm.at[idxp_bufs[b]], inv_bufs[b], isems[b])

        fetch(0, 0)

        def pair_fn(i, _):
            for b in range(2):
                c = i * 2 + b
                nb = 1 - b

                @pl.when(c + 1 < nchunk)
                def _prefetch():
                    @pl.when(c >= 1)
                    def _drain():
                        for s in range(slabs_per_chunk):
                            pltpu.make_async_copy(
                                row_bufs[nb].at[pl.ds(s * l_out, l_out), :],
                                out_hbm.at[s],
                                osems[nb]).wait()

                    fetch(c + 1, nb)

                pltpu.make_async_copy(
                    w_hbm.at[idx_bufs[b]], row_bufs[b], sems[b]).wait()
                pltpu.make_async_copy(
                    inv_hbm.at[idxp_bufs[b]], inv_bufs[b], isems[b]).wait()
                _scale_rows(row_bufs[b], inv_bufs[b], chunk)
                slab0 = (base + c * chunk) // l_out
                for s in range(slabs_per_chunk):
                    pltpu.async_copy(
                        row_bufs[b].at[pl.ds(s * l_out, l_out), :],
                        out_hbm.at[slab0 + s],
                        osems[b])
            return 0

        lax.fori_loop(0, nchunk // 2, pair_fn, 0)
        for b in range(2):
            for s in range(slabs_per_chunk):
                pltpu.make_async_copy(
                    row_bufs[b].at[pl.ds(s * l_out, l_out), :],
                    out_hbm.at[s],
                    osems[b]).wait()

    return sc_kernel


def kernel(x, weight):
    b, l = x.shape
    flat_idx = x.reshape(b * l).astype(jnp.int32)
    inv = _tc_inv_norm(weight)
    half = b // 2
    sc = _make_sc_kernel(half, l, chunk=8 * l)
    out_a = sc(flat_idx[: half * l], weight, inv)
    out_b = sc(flat_idx[half * l:], weight, inv)
    out = jnp.zeros((b, l, _DIM), jnp.float32)
    out = lax.dynamic_update_slice(out, out_a, (0, 0, 0))
    return lax.dynamic_update_slice(out, out_b, (half, 0, 0))


# speedup vs baseline: 1.4194x; 1.4194x over previous
import functools

import jax
import jax.numpy as jnp
from jax import lax
from jax.experimental import pallas as pl
from jax.experimental.pallas import tpu as pltpu
from jax.experimental.pallas import tpu_sc as plsc

_DIM = 128
_LANES = 16
_NC = 2
_NS = 16
_NW = _NC * _NS
_VPR = _DIM // _LANES

_SUB = 10000
_PADBLK = 10240


def _tc_inv_norm(weight):
    v = weight.shape[0]
    assert v % _SUB == 0
    nb = v // _SUB

    def body(w_ref, inv_ref):
        w = w_ref[...]
        ones = jnp.ones((_DIM, 1), jnp.float32)
        ss = lax.dot_general(w * w, ones, (((1,), (0,)), ((), ())),
                             preferred_element_type=jnp.float32)[:, 0]
        inv_ref[pl.ds(0, _SUB)] = lax.rsqrt(jnp.maximum(ss, 1e-24))

    return pl.pallas_call(
        body,
        grid=(nb,),
        in_specs=[pl.BlockSpec((_SUB, _DIM), lambda i: (i, 0))],
        out_specs=pl.BlockSpec((_PADBLK,), lambda i: (i,)),
        out_shape=jax.ShapeDtypeStruct((nb * _PADBLK,), jnp.float32),
    )(weight)


def _scale_rows(rows_ref, inv_ref, nrows):
    @plsc.parallel_loop(0, nrows // _LANES, step=1)
    def group_fn(g):
        iv = inv_ref[pl.ds(g * _LANES, _LANES)]
        for k in range(_LANES):
            r = g * _LANES + k
            inv = iv[k]
            for j in range(_VPR):
                sl = pl.ds(j * _LANES, _LANES)
                rows_ref[r, sl] = rows_ref[r, sl] * inv


def _make_sc_kernel(nb_out, l_out, chunk):
    n_idx = nb_out * l_out
    assert chunk % l_out == 0 and n_idx % (_NW * chunk) == 0
    assert chunk % _LANES == 0
    slabs_per_chunk = chunk // l_out
    per_w = n_idx // _NW
    nchunk = per_w // chunk
    assert nchunk % 2 == 0
    mesh = plsc.VectorSubcoreMesh(core_axis_name="c", subcore_axis_name="s")

    @functools.partial(
        pl.kernel,
        out_type=jax.ShapeDtypeStruct((nb_out, l_out, _DIM), jnp.float32),
        mesh=mesh,
        scratch_types=[
            pltpu.VMEM((per_w,), jnp.int32),
            pltpu.VMEM((per_w,), jnp.int32),
            pltpu.VMEM((chunk, _DIM), jnp.float32),
            pltpu.VMEM((chunk, _DIM), jnp.float32),
            pltpu.VMEM((chunk,), jnp.float32),
            pltpu.VMEM((chunk,), jnp.float32),
            pltpu.SemaphoreType.DMA,
            pltpu.SemaphoreType.DMA,
            pltpu.SemaphoreType.DMA,
            pltpu.SemaphoreType.DMA,
            pltpu.SemaphoreType.DMA,
            pltpu.SemaphoreType.DMA,
        ],
        compiler_params=pltpu.CompilerParams(needs_layout_passes=False,
                                             use_tc_tiling_on_sc=True),
    )
    def sc_kernel(idx_hbm, w_hbm, inv_hbm, out_hbm, idx_all, idxp_all,
                  rows_a, rows_b, inv_a, inv_b, sem_a, sem_b,
                  isem_a, isem_b, osem_a, osem_b):
        wid = lax.axis_index("s") * _NC + lax.axis_index("c")
        base = wid * per_w
        row_bufs = (rows_a, rows_b)
        inv_bufs = (inv_a, inv_b)
        sems = (sem_a, sem_b)
        isems = (isem_a, isem_b)
        osems = (osem_a, osem_b)

        pltpu.sync_copy(idx_hbm.at[pl.ds(base, per_w)], idx_all)

        @plsc.parallel_loop(0, per_w // _LANES, step=1, unroll=4)
        def pad_fn(t):
            sl = pl.ds(t * _LANES, _LANES)
            iv = idx_all[sl]
            idxp_all[sl] = iv + (_PADBLK - _SUB) * (iv // _SUB)

        def fetch(c, b):
            off = c * chunk
            pltpu.async_copy(w_hbm.at[idx_all.at[pl.ds(off, chunk)]],
                             row_bufs[b], sems[b])
            pltpu.async_copy(inv_hbm.at[idxp_all.at[pl.ds(off, chunk)]],
                             inv_bufs[b], isems[b])

        fetch(0, 0)

        def pair_fn(i, _):
            for b in range(2):
                c = i * 2 + b
                nb = 1 - b

                @pl.when(c + 1 < nchunk)
                def _prefetch():
                    @pl.when(c >= 1)
                    def _drain():
                        for s in range(slabs_per_chunk):
                            pltpu.make_async_copy(
                                row_bufs[nb].at[pl.ds(s * l_out, l_out), :],
                                out_hbm.at[s],
                                osems[nb]).wait()

                    fetch(c + 1, nb)

                pltpu.make_async_copy(
                    w_hbm.at[idx_all.at[pl.ds(0, chunk)]],
                    row_bufs[b], sems[b]).wait()
                pltpu.make_async_copy(
                    inv_hbm.at[idxp_all.at[pl.ds(0, chunk)]],
                    inv_bufs[b], isems[b]).wait()
                _scale_rows(row_bufs[b], inv_bufs[b], chunk)
                slab0 = (base + c * chunk) // l_out
                for s in range(slabs_per_chunk):
                    pltpu.async_copy(
                        row_bufs[b].at[pl.ds(s * l_out, l_out), :],
                        out_hbm.at[slab0 + s],
                        osems[b])
            return 0

        lax.fori_loop(0, nchunk // 2, pair_fn, 0)
        for b in range(2):
            for s in range(slabs_per_chunk):
                pltpu.make_async_copy(
                    row_bufs[b].at[pl.ds(s * l_out, l_out), :],
                    out_hbm.at[s],
                    osems[b]).wait()

    return sc_kernel


def kernel(x, weight):
    b, l = x.shape
    flat_idx = x.reshape(b * l).astype(jnp.int32)
    inv = _tc_inv_norm(weight)
    return _make_sc_kernel(b, l, chunk=8 * l)(flat_idx, weight, inv)


# speedup vs baseline: 1.4383x; 1.0133x over previous
import functools

import jax
import jax.numpy as jnp
from jax import lax
from jax.experimental import pallas as pl
from jax.experimental.pallas import tpu as pltpu
from jax.experimental.pallas import tpu_sc as plsc

_DIM = 128
_LANES = 16
_NC = 2
_NS = 16
_NW = _NC * _NS
_VPR = _DIM // _LANES

_SUB = 10000
_PADBLK = 10240


def _tc_inv_norm(weight):
    v = weight.shape[0]
    assert v % _SUB == 0
    nb = v // _SUB

    def body(w_ref, inv_ref):
        w = w_ref[...]
        ones = jnp.ones((_DIM, 1), jnp.float32)
        ss = lax.dot_general(w * w, ones, (((1,), (0,)), ((), ())),
                             preferred_element_type=jnp.float32)[:, 0]
        inv_ref[pl.ds(0, _SUB)] = lax.rsqrt(jnp.maximum(ss, 1e-24))

    return pl.pallas_call(
        body,
        grid=(nb,),
        in_specs=[pl.BlockSpec((_SUB, _DIM), lambda i: (i, 0))],
        out_specs=pl.BlockSpec((_PADBLK,), lambda i: (i,)),
        out_shape=jax.ShapeDtypeStruct((nb * _PADBLK,), jnp.float32),
    )(weight)


def _scale_rows(rows_ref, inv_ref, nrows):
    @plsc.parallel_loop(0, nrows // _LANES, step=1, unroll=2)
    def group_fn(g):
        iv = inv_ref[pl.ds(g * _LANES, _LANES)]
        for k in range(_LANES):
            r = g * _LANES + k
            inv = iv[k]
            for j in range(_VPR):
                sl = pl.ds(j * _LANES, _LANES)
                rows_ref[r, sl] = rows_ref[r, sl] * inv


def _make_sc_kernel(nb_out, l_out, chunk):
    n_idx = nb_out * l_out
    assert chunk % l_out == 0 and n_idx % (_NW * chunk) == 0
    assert chunk % _LANES == 0
    slabs_per_chunk = chunk // l_out
    per_w = n_idx // _NW
    nchunk = per_w // chunk
    assert nchunk % 2 == 0
    mesh = plsc.VectorSubcoreMesh(core_axis_name="c", subcore_axis_name="s")

    @functools.partial(
        pl.kernel,
        out_type=jax.ShapeDtypeStruct((nb_out, l_out, _DIM), jnp.float32),
        mesh=mesh,
        scratch_types=[
            pltpu.VMEM((chunk,), jnp.int32),
            pltpu.VMEM((chunk,), jnp.int32),
            pltpu.VMEM((chunk,), jnp.int32),
            pltpu.VMEM((chunk,), jnp.int32),
            pltpu.VMEM((chunk, _DIM), jnp.float32),
            pltpu.VMEM((chunk, _DIM), jnp.float32),
            pltpu.VMEM((chunk,), jnp.float32),
            pltpu.VMEM((chunk,), jnp.float32),
            pltpu.SemaphoreType.DMA,
            pltpu.SemaphoreType.DMA,
            pltpu.SemaphoreType.DMA,
            pltpu.SemaphoreType.DMA,
            pltpu.SemaphoreType.DMA,
            pltpu.SemaphoreType.DMA,
        ],
        compiler_params=pltpu.CompilerParams(needs_layout_passes=False,
                                             use_tc_tiling_on_sc=True),
    )
    def sc_kernel(idx_hbm, w_hbm, inv_hbm, out_hbm, idx_a, idx_b, idxp_a,
                  idxp_b, rows_a, rows_b, inv_a, inv_b, sem_a, sem_b,
                  isem_a, isem_b, osem_a, osem_b):
        wid = lax.axis_index("s") * _NC + lax.axis_index("c")
        base = wid * per_w
        idx_bufs = (idx_a, idx_b)
        idxp_bufs = (idxp_a, idxp_b)
        row_bufs = (rows_a, rows_b)
        inv_bufs = (inv_a, inv_b)
        sems = (sem_a, sem_b)
        isems = (isem_a, isem_b)
        osems = (osem_a, osem_b)

        def fetch(c, b):
            off = base + c * chunk
            pltpu.sync_copy(idx_hbm.at[pl.ds(off, chunk)], idx_bufs[b])

            @plsc.parallel_loop(0, chunk // _LANES, step=1, unroll=2)
            def pad_fn(t):
                sl = pl.ds(t * _LANES, _LANES)
                iv = idx_bufs[b][sl]
                idxp_bufs[b][sl] = iv + (_PADBLK - _SUB) * (iv // _SUB)

            pltpu.async_copy(w_hbm.at[idx_bufs[b]], row_bufs[b], sems[b])
            pltpu.async_copy(inv_hbm.at[idxp_bufs[b]], inv_bufs[b], isems[b])

        fetch(0, 0)

        def pair_fn(i, _):
            for b in range(2):
                c = i * 2 + b
                nb = 1 - b

                @pl.when(c + 1 < nchunk)
                def _prefetch():
                    @pl.when(c >= 1)
                    def _drain():
                        for s in range(slabs_per_chunk):
                            pltpu.make_async_copy(
                                row_bufs[nb].at[pl.ds(s * l_out, l_out), :],
                                out_hbm.at[s],
                                osems[nb]).wait()

                    fetch(c + 1, nb)

                pltpu.make_async_copy(
                    w_hbm.at[idx_bufs[b]], row_bufs[b], sems[b]).wait()
                pltpu.make_async_copy(
                    inv_hbm.at[idxp_bufs[b]], inv_bufs[b], isems[b]).wait()
                _scale_rows(row_bufs[b], inv_bufs[b], chunk)
                slab0 = (base + c * chunk) // l_out
                for s in range(slabs_per_chunk):
                    pltpu.async_copy(
                        row_bufs[b].at[pl.ds(s * l_out, l_out), :],
                        out_hbm.at[slab0 + s],
                        osems[b])
            return 0

        lax.fori_loop(0, nchunk // 2, pair_fn, 0)
        for b in range(2):
            for s in range(slabs_per_chunk):
                pltpu.make_async_copy(
                    row_bufs[b].at[pl.ds(s * l_out, l_out), :],
                    out_hbm.at[s],
                    osems[b]).wait()

    return sc_kernel


def kernel(x, weight):
    b, l = x.shape
    flat_idx = x.reshape(b * l).astype(jnp.int32)
    inv = _tc_inv_norm(weight)
    return _make_sc_kernel(b, l, chunk=8 * l)(flat_idx, weight, inv)
